# baseline (speedup 1.0000x reference)
import functools

import jax
import jax.numpy as jnp
from jax import lax
from jax.experimental import pallas as pl
from jax.experimental.pallas import tpu as pltpu
from jax.experimental.pallas import tpu_sc as plsc

_N = 10000
_E = 320000
_D = 128
_H = 8
_DH = 16
_R = 4
_OUT = 64
_SLOPE = 0.2
_EPS = 1e-9

_NC = 2
_NS = 16
_HD = _D // _NC
_EPW = _E // _NS
_SW = 80
_SS = 5
_B = _SW * _SS
_CHUNKS = _EPW // _B
_RPT = 624
_TAIL = _N - _RPT * _NS
_ZR = 208


def _proj_body(h_ref, w_ref, g_ref):
    g_ref[...] = jnp.dot(h_ref[...], w_ref[...],
                         preferred_element_type=jnp.float32)


def _project(h, wbig):
    return pl.pallas_call(
        _proj_body,
        out_shape=jax.ShapeDtypeStruct((_N, 2 * _D), jnp.float32),
    )(h, wbig)


def _edge_body(src_hbm, dst_hbm, attr_hbm, qt_hbm, kt_hbm, v0_hbm, v1_hbm,
               u_out, s_out,
               srcv, dstv, idxq, idxk, sdix,
               qrows, krows, vrows,
               sem,
               u_sh, s_sh):
    cid = lax.axis_index("c")
    sid = lax.axis_index("s")
    base_n = sid * _RPT

    def _zb(i, _):
        for j in range(_HD // 16):
            vrows[0, i, pl.ds(j * 16, 16)] = jnp.zeros((16,), jnp.float32)
        qrows[0, i] = jnp.zeros((16,), jnp.float32)
        return 0
    lax.fori_loop(0, _ZR, _zb, 0)

    for jj in range(_RPT // _ZR):
        pltpu.sync_copy(vrows.at[0].at[pl.ds(0, _ZR)],
                        u_sh.at[pl.ds(base_n + jj * _ZR, _ZR)])
        pltpu.sync_copy(qrows.at[0].at[pl.ds(0, _ZR)],
                        s_sh.at[pl.ds(base_n + jj * _ZR, _ZR)])

    @pl.when(sid == _NS - 1)
    def _zero_tail():
        pltpu.sync_copy(vrows.at[0].at[pl.ds(0, _TAIL)],
                        u_sh.at[pl.ds(_RPT * _NS, _TAIL)])
        pltpu.sync_copy(qrows.at[0].at[pl.ds(0, _TAIL)],
                        s_sh.at[pl.ds(_RPT * _NS, _TAIL)])

    plsc.subcore_barrier()

    nrow = _EPW // _SW

    def _fire_loads(g, p):
        rb = sid * nrow + g * _SS
        pltpu.async_copy(src_hbm.at[pl.ds(rb, _SS)], srcv.at[p], sem)
        pltpu.async_copy(dst_hbm.at[pl.ds(rb, _SS)], dstv.at[p], sem)
        pltpu.async_copy(attr_hbm.at[pl.ds(rb, _SS)], idxq.at[p], sem)

    def _drain_loads(p):
        pltpu.make_async_copy(src_hbm.at[pl.ds(0, _SS)], srcv.at[p], sem).wait()
        pltpu.make_async_copy(dst_hbm.at[pl.ds(0, _SS)], dstv.at[p], sem).wait()
        pltpu.make_async_copy(attr_hbm.at[pl.ds(0, _SS)], idxq.at[p], sem).wait()

    def _idx(p):
        for j in range(_SS):
            for i in range(_SW // 16):
                sl = pl.ds(i * 16, 16)
                d = dstv[p, j, sl]
                idxk[p, j, sl] = srcv[p, j, sl] * _R + idxq[p, j, sl]
                idxq[p, j, sl] = d * _R + idxq[p, j, sl]
                sdix[p, j, sl] = d

    def _fire_gathers(p):
        for j in range(_SS):
            rows = pl.ds(j * _SW, _SW)
            pltpu.async_copy(qt_hbm.at[idxq.at[p, j]], qrows.at[p].at[rows], sem)
            pltpu.async_copy(kt_hbm.at[idxk.at[p, j]], krows.at[p].at[rows], sem)

        @pl.when(cid == 0)
        def _gv0():
            for j in range(_SS):
                pltpu.async_copy(v0_hbm.at[srcv.at[p, j]],
                                 vrows.at[p].at[pl.ds(j * _SW, _SW)], sem)

        @pl.when(cid == 1)
        def _gv1():
            for j in range(_SS):
                pltpu.async_copy(v1_hbm.at[srcv.at[p, j]],
                                 vrows.at[p].at[pl.ds(j * _SW, _SW)], sem)

    def _drain_gathers(p):
        for j in range(_SS):
            rows = pl.ds(j * _SW, _SW)
            pltpu.make_async_copy(qt_hbm.at[idxq.at[p, j]],
                                  qrows.at[p].at[rows], sem).wait()
            pltpu.make_async_copy(kt_hbm.at[idxk.at[p, j]],
                                  krows.at[p].at[rows], sem).wait()
            pltpu.make_async_copy(v0_hbm.at[srcv.at[p, j]],
                                  vrows.at[p].at[rows], sem).wait()

    def _compute_and_scatter(p, ex_on):
        def _mk_cm(lane0):
            def _cm(b, _):
                e = qrows[p, b] + krows[p, b]
                e = jnp.where(e >= 0.0, e, e * _SLOPE)
                ex = jnp.exp(e)
                krows[p, b] = ex
                for hh in range(_H // _NC):
                    sl = pl.ds(hh * _DH, _DH)
                    vrows[p, b, sl] = vrows[p, b, sl] * ex[lane0 + hh]
                return 0
            return _cm

        @pl.when(cid == 0)
        def _m0():
            lax.fori_loop(0, _B, _mk_cm(0), 0, unroll=8)

        @pl.when(cid == 1)
        def _m1():
            lax.fori_loop(0, _B, _mk_cm(_H // _NC), 0, unroll=8)

        @pl.when(ex_on)
        def _sx():
            for j in range(_SS):
                pltpu.async_copy(krows.at[p].at[pl.ds(j * _SW, _SW)],
                                 s_sh.at[sdix.at[p, j]], sem, add=True)

        for j in range(_SS):
            pltpu.async_copy(vrows.at[p].at[pl.ds(j * _SW, _SW)],
                             u_sh.at[sdix.at[p, j]], sem, add=True)

    def _drain_scatters(p, ex_on):
        @pl.when(ex_on)
        def _dx():
            for j in range(_SS):
                pltpu.make_async_copy(krows.at[p].at[pl.ds(j * _SW, _SW)],
                                      s_sh.at[sdix.at[p, j]], sem).wait()
        for j in range(_SS):
            pltpu.make_async_copy(vrows.at[p].at[pl.ds(j * _SW, _SW)],
                                  u_sh.at[sdix.at[p, j]], sem).wait()

    def _chunk(g, _):
        ex_on = jnp.where(cid == 0, g < _CHUNKS // 2, g >= _CHUNKS // 2)
        _fire_loads(g, 0)
        _drain_loads(0)
        _idx(0)
        _fire_gathers(0)
        _drain_gathers(0)
        _compute_and_scatter(0, ex_on)
        _drain_scatters(0, ex_on)
        return 0

    lax.fori_loop(0, _CHUNKS, _chunk, 0)
    plsc.subcore_barrier()

    for jj in range(_RPT // _ZR):
        rb = base_n + jj * _ZR
        pltpu.sync_copy(u_sh.at[pl.ds(rb, _ZR)], vrows.at[0].at[pl.ds(0, _ZR)])
        pltpu.sync_copy(vrows.at[0].at[pl.ds(0, _ZR)],
                        u_out.at[pl.ds(cid * _N + rb, _ZR)])
        pltpu.sync_copy(s_sh.at[pl.ds(rb, _ZR)], qrows.at[0].at[pl.ds(0, _ZR)])
        pltpu.sync_copy(qrows.at[0].at[pl.ds(0, _ZR)],
                        s_out.at[pl.ds(cid * _N + rb, _ZR)])

    @pl.when(sid == _NS - 1)
    def _dump_tail():
        tb = _RPT * _NS
        pltpu.sync_copy(u_sh.at[pl.ds(tb, _TAIL)], vrows.at[0].at[pl.ds(0, _TAIL)])
        pltpu.sync_copy(vrows.at[0].at[pl.ds(0, _TAIL)],
                        u_out.at[pl.ds(cid * _N + tb, _TAIL)])
        pltpu.sync_copy(s_sh.at[pl.ds(tb, _TAIL)], qrows.at[0].at[pl.ds(0, _TAIL)])
        pltpu.sync_copy(qrows.at[0].at[pl.ds(0, _TAIL)],
                        s_out.at[pl.ds(cid * _N + tb, _TAIL)])


_edge_kernel = functools.partial(
    pl.kernel,
    out_type=[
        jax.ShapeDtypeStruct((_NC * _N, _HD), jnp.float32),
        jax.ShapeDtypeStruct((_NC * _N, 16), jnp.float32),
    ],
    mesh=plsc.VectorSubcoreMesh(core_axis_name="c", subcore_axis_name="s"),
    compiler_params=pltpu.CompilerParams(use_tc_tiling_on_sc=False),
    scratch_types=[
        pltpu.VMEM((2, _SS, _SW), jnp.int32),
        pltpu.VMEM((2, _SS, _SW), jnp.int32),
        pltpu.VMEM((2, _SS, _SW), jnp.int32),
        pltpu.VMEM((2, _SS, _SW), jnp.int32),
        pltpu.VMEM((2, _SS, _SW), jnp.int32),
        pltpu.VMEM((2, _B, 16), jnp.float32),
        pltpu.VMEM((2, _B, 16), jnp.float32),
        pltpu.VMEM((2, _B, _HD), jnp.float32),
        pltpu.SemaphoreType.DMA,
        pltpu.VMEM_SHARED((_N, _HD), jnp.float32),
        pltpu.VMEM_SHARED((_N, 16), jnp.float32),
    ],
)(_edge_body)


def _final_body(u_ref, s_ref, h_ref, wo_ref, wp_ref, m_ref, o_ref):
    u = jnp.concatenate([u_ref[: _N], u_ref[_N:]], axis=1)
    s = s_ref[: _N] + s_ref[_N:]
    denom = jnp.dot(s[:, :_H], m_ref[...],
                    preferred_element_type=jnp.float32) + _EPS
    agg = u / denom
    w1 = jnp.dot(wo_ref[...], wp_ref[...], preferred_element_type=jnp.float32)
    o_ref[...] = (jnp.dot(agg, w1, preferred_element_type=jnp.float32)
                  + jnp.dot(h_ref[...], wp_ref[...],
                            preferred_element_type=jnp.float32))


def _finalize(u, s, h, wo, wp, m):
    return pl.pallas_call(
        _final_body,
        out_shape=jax.ShapeDtypeStruct((_N, _OUT), jnp.float32),
    )(u, s, h, wo, wp, m)


def kernel(emb, edge_index, edge_attr, node_type, local_node_idx,
           Wq, Wk, Wv, a_rel, Wo, Wproj):
    f32 = jnp.float32
    h = jnp.take(emb, local_node_idx, axis=0)

    aq = jnp.einsum('dhe,rhe->drh', Wq.reshape(_D, _H, _DH), a_rel)
    ak = jnp.einsum('dhe,rhe->drh', Wk.reshape(_D, _H, _DH), a_rel)
    pad = jnp.zeros((_D, _R, 16 - _H), f32)
    wqa = jnp.concatenate([aq, pad], axis=-1).reshape(_D, _R * 16)
    wka = jnp.concatenate([ak, pad], axis=-1).reshape(_D, _R * 16)
    wbig = jnp.concatenate([Wv, wqa, wka], axis=1)

    g = _project(h, wbig)
    v_tab = g[:, :_D]
    qt = g[:, _D:_D + 64].reshape(_N * _R, 16)
    kt = g[:, _D + 64:].reshape(_N * _R, 16)

    src = edge_index[0].reshape(_E // _SW, _SW)
    dst = edge_index[1].reshape(_E // _SW, _SW)
    attr2 = edge_attr.reshape(_E // _SW, _SW)
    v0 = v_tab[:, :_HD]
    v1 = v_tab[:, _HD:]
    u, s = _edge_kernel(src, dst, attr2, qt, kt, v0, v1)

    m = jnp.kron(jnp.eye(_H, dtype=f32), jnp.ones((1, _DH), f32))
    return _finalize(u, s, h, Wo, Wproj, m)


# speedup vs baseline: 1.0086x; 1.0086x over previous
import functools

import jax
import jax.numpy as jnp
from jax import lax
from jax.experimental import pallas as pl
from jax.experimental.pallas import tpu as pltpu
from jax.experimental.pallas import tpu_sc as plsc

_N = 10000
_E = 320000
_D = 128
_H = 8
_DH = 16
_R = 4
_OUT = 64
_SLOPE = 0.2
_EPS = 1e-9

_NC = 2
_NS = 16
_HD = _D // _NC
_EPW = _E // _NS
_SW = 80
_SS = 5
_B = _SW * _SS
_CHUNKS = _EPW // _B
_RPT = 624
_TAIL = _N - _RPT * _NS
_ZR = 208


def _proj_body(h_ref, w_ref, g_ref):
    g_ref[...] = jnp.dot(h_ref[...], w_ref[...],
                         preferred_element_type=jnp.float32)


def _project(h, wbig):
    return pl.pallas_call(
        _proj_body,
        out_shape=jax.ShapeDtypeStruct((_N, 2 * _D), jnp.float32),
    )(h, wbig)


def _edge_body(src_hbm, dst_hbm, attr_hbm, qt_hbm, kt_hbm, v0_hbm, v1_hbm,
               u_out, s_out,
               srcv, dstv, idxq, idxk, sdix,
               qrows, krows, vrows,
               sem,
               u_sh, s_sh):
    cid = lax.axis_index("c")
    sid = lax.axis_index("s")
    base_n = sid * _RPT

    def _zb(i, _):
        for j in range(_HD // 16):
            vrows[0, i, pl.ds(j * 16, 16)] = jnp.zeros((16,), jnp.float32)
        qrows[0, i] = jnp.zeros((16,), jnp.float32)
        return 0
    lax.fori_loop(0, _ZR, _zb, 0)

    for jj in range(_RPT // _ZR):
        pltpu.sync_copy(vrows.at[0].at[pl.ds(0, _ZR)],
                        u_sh.at[pl.ds(base_n + jj * _ZR, _ZR)])
        pltpu.sync_copy(qrows.at[0].at[pl.ds(0, _ZR)],
                        s_sh.at[pl.ds(base_n + jj * _ZR, _ZR)])

    @pl.when(sid == _NS - 1)
    def _zero_tail():
        pltpu.sync_copy(vrows.at[0].at[pl.ds(0, _TAIL)],
                        u_sh.at[pl.ds(_RPT * _NS, _TAIL)])
        pltpu.sync_copy(qrows.at[0].at[pl.ds(0, _TAIL)],
                        s_sh.at[pl.ds(_RPT * _NS, _TAIL)])

    plsc.subcore_barrier()

    nrow = _EPW // _SW

    def _fire_loads(g, p):
        rb = sid * nrow + g * _SS
        pltpu.async_copy(src_hbm.at[pl.ds(rb, _SS)], srcv.at[p], sem)
        pltpu.async_copy(dst_hbm.at[pl.ds(rb, _SS)], dstv.at[p], sem)
        pltpu.async_copy(attr_hbm.at[pl.ds(rb, _SS)], idxq.at[p], sem)

    def _drain_loads(p):
        pltpu.make_async_copy(src_hbm.at[pl.ds(0, _SS)], srcv.at[p], sem).wait()
        pltpu.make_async_copy(dst_hbm.at[pl.ds(0, _SS)], dstv.at[p], sem).wait()
        pltpu.make_async_copy(attr_hbm.at[pl.ds(0, _SS)], idxq.at[p], sem).wait()

    def _idx(p):
        for j in range(_SS):
            for i in range(_SW // 16):
                sl = pl.ds(i * 16, 16)
                d = dstv[p, j, sl]
                idxk[p, j, sl] = srcv[p, j, sl] * _R + idxq[p, j, sl]
                idxq[p, j, sl] = d * _R + idxq[p, j, sl]
                sdix[p, j, sl] = d

    def _fire_gathers(p):
        for j in range(_SS):
            rows = pl.ds(j * _SW, _SW)
            pltpu.async_copy(qt_hbm.at[idxq.at[p, j]], qrows.at[p].at[rows], sem)
            pltpu.async_copy(kt_hbm.at[idxk.at[p, j]], krows.at[p].at[rows], sem)

        @pl.when(cid == 0)
        def _gv0():
            for j in range(_SS):
                pltpu.async_copy(v0_hbm.at[srcv.at[p, j]],
                                 vrows.at[p].at[pl.ds(j * _SW, _SW)], sem)

        @pl.when(cid == 1)
        def _gv1():
            for j in range(_SS):
                pltpu.async_copy(v1_hbm.at[srcv.at[p, j]],
                                 vrows.at[p].at[pl.ds(j * _SW, _SW)], sem)

    def _drain_gathers(p):
        for j in range(_SS):
            rows = pl.ds(j * _SW, _SW)
            pltpu.make_async_copy(qt_hbm.at[idxq.at[p, j]],
                                  qrows.at[p].at[rows], sem).wait()
            pltpu.make_async_copy(kt_hbm.at[idxk.at[p, j]],
                                  krows.at[p].at[rows], sem).wait()
            pltpu.make_async_copy(v0_hbm.at[srcv.at[p, j]],
                                  vrows.at[p].at[rows], sem).wait()

    def _compute_and_scatter(p, ex_on):
        def _mk_cm(lane0):
            def _cm(b, _):
                e = qrows[p, b] + krows[p, b]
                e = jnp.where(e >= 0.0, e, e * _SLOPE)
                ex = jnp.exp(e)
                krows[p, b] = ex
                for hh in range(_H // _NC):
                    sl = pl.ds(hh * _DH, _DH)
                    vrows[p, b, sl] = vrows[p, b, sl] * ex[lane0 + hh]
                return 0
            return _cm

        @pl.when(cid == 0)
        def _m0():
            lax.fori_loop(0, _B, _mk_cm(0), 0, unroll=2)

        @pl.when(cid == 1)
        def _m1():
            lax.fori_loop(0, _B, _mk_cm(_H // _NC), 0, unroll=2)

        @pl.when(ex_on)
        def _sx():
            for j in range(_SS):
                pltpu.async_copy(krows.at[p].at[pl.ds(j * _SW, _SW)],
                                 s_sh.at[sdix.at[p, j]], sem, add=True)

        for j in range(_SS):
            pltpu.async_copy(vrows.at[p].at[pl.ds(j * _SW, _SW)],
                             u_sh.at[sdix.at[p, j]], sem, add=True)

    def _drain_scatters(p, ex_on):
        @pl.when(ex_on)
        def _dx():
            for j in range(_SS):
                pltpu.make_async_copy(krows.at[p].at[pl.ds(j * _SW, _SW)],
                                      s_sh.at[sdix.at[p, j]], sem).wait()
        for j in range(_SS):
            pltpu.make_async_copy(vrows.at[p].at[pl.ds(j * _SW, _SW)],
                                  u_sh.at[sdix.at[p, j]], sem).wait()

    def _chunk(g, _):
        ex_on = jnp.where(cid == 0, g < _CHUNKS // 2, g >= _CHUNKS // 2)
        _fire_loads(g, 0)
        _drain_loads(0)
        _idx(0)
        _fire_gathers(0)
        _drain_gathers(0)
        _compute_and_scatter(0, ex_on)
        _drain_scatters(0, ex_on)
        return 0

    lax.fori_loop(0, _CHUNKS, _chunk, 0)
    plsc.subcore_barrier()

    for jj in range(_RPT // _ZR):
        rb = base_n + jj * _ZR
        pltpu.sync_copy(u_sh.at[pl.ds(rb, _ZR)], vrows.at[0].at[pl.ds(0, _ZR)])
        pltpu.sync_copy(vrows.at[0].at[pl.ds(0, _ZR)],
                        u_out.at[pl.ds(cid * _N + rb, _ZR)])
        pltpu.sync_copy(s_sh.at[pl.ds(rb, _ZR)], qrows.at[0].at[pl.ds(0, _ZR)])
        pltpu.sync_copy(qrows.at[0].at[pl.ds(0, _ZR)],
                        s_out.at[pl.ds(cid * _N + rb, _ZR)])

    @pl.when(sid == _NS - 1)
    def _dump_tail():
        tb = _RPT * _NS
        pltpu.sync_copy(u_sh.at[pl.ds(tb, _TAIL)], vrows.at[0].at[pl.ds(0, _TAIL)])
        pltpu.sync_copy(vrows.at[0].at[pl.ds(0, _TAIL)],
                        u_out.at[pl.ds(cid * _N + tb, _TAIL)])
        pltpu.sync_copy(s_sh.at[pl.ds(tb, _TAIL)], qrows.at[0].at[pl.ds(0, _TAIL)])
        pltpu.sync_copy(qrows.at[0].at[pl.ds(0, _TAIL)],
                        s_out.at[pl.ds(cid * _N + tb, _TAIL)])


_edge_kernel = functools.partial(
    pl.kernel,
    out_type=[
        jax.ShapeDtypeStruct((_NC * _N, _HD), jnp.float32),
        jax.ShapeDtypeStruct((_NC * _N, 16), jnp.float32),
    ],
    mesh=plsc.VectorSubcoreMesh(core_axis_name="c", subcore_axis_name="s"),
    compiler_params=pltpu.CompilerParams(use_tc_tiling_on_sc=False),
    scratch_types=[
        pltpu.VMEM((2, _SS, _SW), jnp.int32),
        pltpu.VMEM((2, _SS, _SW), jnp.int32),
        pltpu.VMEM((2, _SS, _SW), jnp.int32),
        pltpu.VMEM((2, _SS, _SW), jnp.int32),
        pltpu.VMEM((2, _SS, _SW), jnp.int32),
        pltpu.VMEM((2, _B, 16), jnp.float32),
        pltpu.VMEM((2, _B, 16), jnp.float32),
        pltpu.VMEM((2, _B, _HD), jnp.float32),
        pltpu.SemaphoreType.DMA,
        pltpu.VMEM_SHARED((_N, _HD), jnp.float32),
        pltpu.VMEM_SHARED((_N, 16), jnp.float32),
    ],
)(_edge_body)


def _final_body(u_ref, s_ref, h_ref, wo_ref, wp_ref, m_ref, o_ref):
    u = jnp.concatenate([u_ref[: _N], u_ref[_N:]], axis=1)
    s = s_ref[: _N] + s_ref[_N:]
    denom = jnp.dot(s[:, :_H], m_ref[...],
                    preferred_element_type=jnp.float32) + _EPS
    agg = u / denom
    w1 = jnp.dot(wo_ref[...], wp_ref[...], preferred_element_type=jnp.float32)
    o_ref[...] = (jnp.dot(agg, w1, preferred_element_type=jnp.float32)
                  + jnp.dot(h_ref[...], wp_ref[...],
                            preferred_element_type=jnp.float32))


def _finalize(u, s, h, wo, wp, m):
    return pl.pallas_call(
        _final_body,
        out_shape=jax.ShapeDtypeStruct((_N, _OUT), jnp.float32),
    )(u, s, h, wo, wp, m)


def kernel(emb, edge_index, edge_attr, node_type, local_node_idx,
           Wq, Wk, Wv, a_rel, Wo, Wproj):
    f32 = jnp.float32
    h = jnp.take(emb, local_node_idx, axis=0)

    aq = jnp.einsum('dhe,rhe->drh', Wq.reshape(_D, _H, _DH), a_rel)
    ak = jnp.einsum('dhe,rhe->drh', Wk.reshape(_D, _H, _DH), a_rel)
    pad = jnp.zeros((_D, _R, 16 - _H), f32)
    wqa = jnp.concatenate([aq, pad], axis=-1).reshape(_D, _R * 16)
    wka = jnp.concatenate([ak, pad], axis=-1).reshape(_D, _R * 16)
    wbig = jnp.concatenate([Wv, wqa, wka], axis=1)

    g = _project(h, wbig)
    v_tab = g[:, :_D]
    qt = g[:, _D:_D + 64].reshape(_N * _R, 16)
    kt = g[:, _D + 64:].reshape(_N * _R, 16)

    src = edge_index[0].reshape(_E // _SW, _SW)
    dst = edge_index[1].reshape(_E // _SW, _SW)
    attr2 = edge_attr.reshape(_E // _SW, _SW)
    v0 = v_tab[:, :_HD]
    v1 = v_tab[:, _HD:]
    u, s = _edge_kernel(src, dst, attr2, qt, kt, v0, v1)

    m = jnp.kron(jnp.eye(_H, dtype=f32), jnp.ones((1, _DH), f32))
    return _finalize(u, s, h, Wo, Wproj, m)


# speedup vs baseline: 1.3156x; 1.3043x over previous
import functools

import jax
import jax.numpy as jnp
from jax import lax
from jax.experimental import pallas as pl
from jax.experimental.pallas import tpu as pltpu
from jax.experimental.pallas import tpu_sc as plsc

_N = 10000
_E = 320000
_D = 128
_H = 8
_DH = 16
_R = 4
_OUT = 64
_SLOPE = 0.2
_EPS = 1e-9

_NC = 2
_NS = 16
_HD = _D // _NC
_EPW = _E // _NS
_SW = 80
_SS = 5
_B = _SW * _SS
_CHUNKS = _EPW // _B
_RPT = 624
_TAIL = _N - _RPT * _NS
_ZR = 208


def _proj_body(h_ref, w_ref, g_ref):
    g_ref[...] = jnp.dot(h_ref[...], w_ref[...],
                         preferred_element_type=jnp.float32)


def _project(h, wbig):
    return pl.pallas_call(
        _proj_body,
        out_shape=jax.ShapeDtypeStruct((_N, 2 * _D), jnp.float32),
    )(h, wbig)


def _edge_body(src_hbm, dst_hbm, attr_hbm, qt_hbm, kt_hbm, v0_hbm, v1_hbm,
               u_out, s_out,
               srcv, dstv, attrv, idxq, idxk,
               qrows, krows, exrows, vrows, zbuf, sbuf, sem,
               u_sh, s_sh):
    cid = lax.axis_index("c")
    sid = lax.axis_index("s")
    base_n = sid * _RPT

    def _zb(i, _):
        for j in range(_HD // 16):
            zbuf[i, pl.ds(j * 16, 16)] = jnp.zeros((16,), jnp.float32)
        return 0
    lax.fori_loop(0, _ZR, _zb, 0)

    def _sb(i, _):
        sbuf[i] = jnp.zeros((16,), jnp.float32)
        return 0
    lax.fori_loop(0, _RPT, _sb, 0)

    for jj in range(_RPT // _ZR):
        pltpu.sync_copy(zbuf, u_sh.at[pl.ds(base_n + jj * _ZR, _ZR)])
    pltpu.sync_copy(sbuf, s_sh.at[pl.ds(base_n, _RPT)])

    @pl.when(sid == _NS - 1)
    def _zero_tail():
        pltpu.sync_copy(zbuf.at[pl.ds(0, _TAIL)],
                        u_sh.at[pl.ds(_RPT * _NS, _TAIL)])
        pltpu.sync_copy(sbuf.at[pl.ds(0, _TAIL)],
                        s_sh.at[pl.ds(_RPT * _NS, _TAIL)])

    plsc.subcore_barrier()

    def _chunk(g, _):
        rb = sid * (_EPW // _SW) + g * _SS
        c1 = pltpu.async_copy(src_hbm.at[pl.ds(rb, _SS)], srcv, sem)
        c2 = pltpu.async_copy(dst_hbm.at[pl.ds(rb, _SS)], dstv, sem)
        c3 = pltpu.async_copy(attr_hbm.at[pl.ds(rb, _SS)], attrv, sem)
        c1.wait(); c2.wait(); c3.wait()

        for j in range(_SS):
            for i in range(_SW // 16):
                sl = pl.ds(i * 16, 16)
                idxq[j, sl] = dstv[j, sl] * _R + attrv[j, sl]
                idxk[j, sl] = srcv[j, sl] * _R + attrv[j, sl]

        gs = []
        for j in range(_SS):
            rows = pl.ds(j * _SW, _SW)
            gs.append(pltpu.async_copy(qt_hbm.at[idxq.at[j]], qrows.at[rows], sem))
            gs.append(pltpu.async_copy(kt_hbm.at[idxk.at[j]], krows.at[rows], sem))

        @pl.when(cid == 0)
        def _gv0():
            hs = [pltpu.async_copy(v0_hbm.at[srcv.at[j]],
                                   vrows.at[pl.ds(j * _SW, _SW)], sem)
                  for j in range(_SS)]
            for h_ in hs:
                h_.wait()

        @pl.when(cid == 1)
        def _gv1():
            hs = [pltpu.async_copy(v1_hbm.at[srcv.at[j]],
                                   vrows.at[pl.ds(j * _SW, _SW)], sem)
                  for j in range(_SS)]
            for h_ in hs:
                h_.wait()

        for g_ in gs:
            g_.wait()

        def _ex(b, _):
            e = qrows[b] + krows[b]
            e = jnp.where(e >= 0.0, e, e * _SLOPE)
            exrows[b] = jnp.exp(e)
            return 0
        lax.fori_loop(0, _B, _ex, 0)

        ss = [pltpu.async_copy(exrows.at[pl.ds(j * _SW, _SW)],
                               s_sh.at[dstv.at[j]], sem, add=True)
              for j in range(_SS)]

        def _mk_msg(lane0):
            def _msg(b, _):
                ex = exrows[b]
                for hh in range(_H // _NC):
                    sl = pl.ds(hh * _DH, _DH)
                    vrows[b, sl] = vrows[b, sl] * ex[lane0 + hh]
                return 0
            return _msg

        @pl.when(cid == 0)
        def _m0():
            lax.fori_loop(0, _B, _mk_msg(0), 0)

        @pl.when(cid == 1)
        def _m1():
            lax.fori_loop(0, _B, _mk_msg(_H // _NC), 0)

        ss.extend(pltpu.async_copy(vrows.at[pl.ds(j * _SW, _SW)],
                                   u_sh.at[dstv.at[j]], sem, add=True)
                  for j in range(_SS))
        for s_ in ss:
            s_.wait()
        return 0

    lax.fori_loop(0, _CHUNKS, _chunk, 0)
    plsc.subcore_barrier()

    for jj in range(_RPT // _ZR):
        rb = base_n + jj * _ZR
        pltpu.sync_copy(u_sh.at[pl.ds(rb, _ZR)], zbuf)
        pltpu.sync_copy(zbuf, u_out.at[pl.ds(cid * _N + rb, _ZR)])
    pltpu.sync_copy(s_sh.at[pl.ds(base_n, _RPT)], sbuf)
    pltpu.sync_copy(sbuf, s_out.at[pl.ds(cid * _N + base_n, _RPT)])

    @pl.when(sid == _NS - 1)
    def _dump_tail():
        tb = _RPT * _NS
        pltpu.sync_copy(u_sh.at[pl.ds(tb, _TAIL)], zbuf.at[pl.ds(0, _TAIL)])
        pltpu.sync_copy(zbuf.at[pl.ds(0, _TAIL)],
                        u_out.at[pl.ds(cid * _N + tb, _TAIL)])
        pltpu.sync_copy(s_sh.at[pl.ds(tb, _TAIL)], sbuf.at[pl.ds(0, _TAIL)])
        pltpu.sync_copy(sbuf.at[pl.ds(0, _TAIL)],
                        s_out.at[pl.ds(cid * _N + tb, _TAIL)])


_edge_kernel = functools.partial(
    pl.kernel,
    out_type=[
        jax.ShapeDtypeStruct((_NC * _N, _HD), jnp.float32),
        jax.ShapeDtypeStruct((_NC * _N, 16), jnp.float32),
    ],
    mesh=plsc.VectorSubcoreMesh(core_axis_name="c", subcore_axis_name="s"),
    compiler_params=pltpu.CompilerParams(use_tc_tiling_on_sc=False),
    scratch_types=[
        pltpu.VMEM((_SS, _SW), jnp.int32),
        pltpu.VMEM((_SS, _SW), jnp.int32),
        pltpu.VMEM((_SS, _SW), jnp.int32),
        pltpu.VMEM((_SS, _SW), jnp.int32),
        pltpu.VMEM((_SS, _SW), jnp.int32),
        pltpu.VMEM((_B, 16), jnp.float32),
        pltpu.VMEM((_B, 16), jnp.float32),
        pltpu.VMEM((_B, 16), jnp.float32),
        pltpu.VMEM((_B, _HD), jnp.float32),
        pltpu.VMEM((_ZR, _HD), jnp.float32),
        pltpu.VMEM((_RPT, 16), jnp.float32),
        pltpu.SemaphoreType.DMA,
        pltpu.VMEM_SHARED((_N, _HD), jnp.float32),
        pltpu.VMEM_SHARED((_N, 16), jnp.float32),
    ],
)(_edge_body)


def _final_body(u_ref, s_ref, h_ref, wo_ref, wp_ref, m_ref, o_ref):
    u = jnp.concatenate([u_ref[: _N], u_ref[_N:]], axis=1)
    s = s_ref[: _N]
    denom = jnp.dot(s[:, :_H], m_ref[...],
                    preferred_element_type=jnp.float32) + _EPS
    agg = u / denom
    w1 = jnp.dot(wo_ref[...], wp_ref[...], preferred_element_type=jnp.float32)
    o_ref[...] = (jnp.dot(agg, w1, preferred_element_type=jnp.float32)
                  + jnp.dot(h_ref[...], wp_ref[...],
                            preferred_element_type=jnp.float32))


def _finalize(u, s, h, wo, wp, m):
    return pl.pallas_call(
        _final_body,
        out_shape=jax.ShapeDtypeStruct((_N, _OUT), jnp.float32),
    )(u, s, h, wo, wp, m)


def kernel(emb, edge_index, edge_attr, node_type, local_node_idx,
           Wq, Wk, Wv, a_rel, Wo, Wproj):
    f32 = jnp.float32
    h = jnp.take(emb, local_node_idx, axis=0)

    aq = jnp.einsum('dhe,rhe->drh', Wq.reshape(_D, _H, _DH), a_rel)
    ak = jnp.einsum('dhe,rhe->drh', Wk.reshape(_D, _H, _DH), a_rel)
    pad = jnp.zeros((_D, _R, 16 - _H), f32)
    wqa = jnp.concatenate([aq, pad], axis=-1).reshape(_D, _R * 16)
    wka = jnp.concatenate([ak, pad], axis=-1).reshape(_D, _R * 16)
    wbig = jnp.concatenate([Wv, wqa, wka], axis=1)

    g = _project(h, wbig)
    v_tab = g[:, :_D]
    qt = g[:, _D:_D + 64].reshape(_N * _R, 16)
    kt = g[:, _D + 64:].reshape(_N * _R, 16)

    src = edge_index[0].reshape(_E // _SW, _SW)
    dst = edge_index[1].reshape(_E // _SW, _SW)
    attr2 = edge_attr.reshape(_E // _SW, _SW)
    v0 = v_tab[:, :_HD]
    v1 = v_tab[:, _HD:]
    u, s = _edge_kernel(src, dst, attr2, qt, kt, v0, v1)

    m = jnp.kron(jnp.eye(_H, dtype=f32), jnp.ones((1, _DH), f32))
    return _finalize(u, s, h, Wo, Wproj, m)
